# baseline (speedup 1.0000x reference)
import functools

import jax
import jax.numpy as jnp
import numpy as np
from jax import lax
from jax.experimental import pallas as pl
from jax.experimental.pallas import tpu as pltpu
from jax.experimental.pallas import tpu_sc as plsc

_B, _S, _D = 4, 512, 128
_Q = _B * _S
_M = 100000
_K = 8
_DH = _D // 2
_KB = 1024
_NB = 98
_MP = _NB * _KB
_G = 128
_GPB = _KB // _G
_NG = _MP // _G
_NGP = 896
_QB4 = 256
_NW = 32

_HIGH = lax.Precision.HIGHEST


def _k0_body(x_ref, wq_ref, qn_ref):
    q = lax.dot_general(x_ref[...], wq_ref[...],
                        (((1,), (1,)), ((), ())),
                        precision=_HIGH, preferred_element_type=jnp.float32)
    ss = jnp.sum(q * q, axis=-1, keepdims=True)
    qn_ref[...] = (q / jnp.maximum(jnp.sqrt(ss), 1e-12)).astype(jnp.bfloat16)


def _k1_body(qn_ref, kb_ref, sims_ref, gmax_ref):
    i = pl.program_id(0)
    kb = kb_ref[...]
    ss = jnp.sum(kb * kb, axis=-1, keepdims=True)
    kn = (kb / jnp.maximum(jnp.sqrt(ss), 1e-12)).astype(jnp.bfloat16)
    s = lax.dot_general(qn_ref[...], kn, (((1,), (1,)), ((), ())),
                        preferred_element_type=jnp.float32)

    def finish(sv):
        chunks = []
        for c in range(_GPB):
            sc = sv[:, c * _G:(c + 1) * _G]
            sims_ref[:, c, :] = sc
            chunks.append(jnp.max(sc, axis=-1, keepdims=True))
        gmax_ref[0] = jnp.concatenate(chunks, axis=-1)

    @pl.when(i < _NB - 1)
    def _():
        finish(s)

    @pl.when(i == _NB - 1)
    def _():
        gcol = i * _KB + lax.broadcasted_iota(jnp.int32, (_Q, _KB), 1)
        finish(jnp.where(gcol < _M, s, jnp.float32(-2.0)))


def _k2_body(g_ref, pid_ref, flat_ref):
    s = g_ref[...].astype(jnp.float32)
    col = lax.broadcasted_iota(jnp.int32, (_Q, _NGP), 1)
    row = lax.broadcasted_iota(jnp.int32, (_Q, _K), 0)
    ids = []
    for _ in range(_K):
        m = jnp.max(s, axis=-1, keepdims=True)
        cand = jnp.where(s >= m, col, jnp.int32(2 ** 30))
        a = jnp.min(cand, axis=-1, keepdims=True)
        ids.append(a)
        s = jnp.where(col == a, jnp.float32(-3.0), s)
    gid = jnp.concatenate(ids, axis=-1)
    pid_ref[...] = gid
    flat_ref[...] = row * _NG + gid


def _k4_body(gs_ref, pid_ref, w_ref, idx_ref):
    s3 = gs_ref[...]
    lane = lax.broadcasted_iota(jnp.int32, (_QB4, _K, _G), 2)
    cidx = pid_ref[...][:, :, None] * _G + lane
    vals = []
    idxs = []
    for _ in range(_K):
        m1 = jnp.max(s3, axis=-1)
        mj = jnp.max(m1, axis=-1, keepdims=True)
        mb = mj[:, :, None]
        cand = jnp.where(s3 >= mb, cidx, jnp.int32(2 ** 30))
        a1 = jnp.min(cand, axis=-1)
        a = jnp.min(a1, axis=-1, keepdims=True)
        vals.append(mj)
        idxs.append(a)
        s3 = jnp.where(cidx == a[:, :, None], jnp.float32(-3.0), s3)
    ts = jnp.concatenate(vals, axis=-1)
    mx = jnp.max(ts, axis=-1, keepdims=True)
    e = jnp.exp(ts - mx)
    w_ref[...] = e / jnp.sum(e, axis=-1, keepdims=True)
    idx_ref[...] = jnp.concatenate(idxs, axis=-1)


def _k6_body(x_ref, v_ref, w_ref, wo_ref, wg1_ref, bg1_ref, wg2_ref,
             o_ref):
    x = x_ref[...]
    w = w_ref[...]
    r = w[:, 0:1] * v_ref[:, 0, :]
    for j in range(1, _K):
        r = r + w[:, j:j + 1] * v_ref[:, j, :]
    retrieved = lax.dot_general(r, wo_ref[...], (((1,), (1,)), ((), ())),
                                precision=_HIGH,
                                preferred_element_type=jnp.float32)
    gate_in = jnp.concatenate([x, retrieved], axis=-1)
    h = lax.dot_general(gate_in, wg1_ref[...], (((1,), (1,)), ((), ())),
                        precision=_HIGH, preferred_element_type=jnp.float32)
    h = h + bg1_ref[...]
    h = 0.5 * h * (1.0 + lax.erf(h * np.float32(1.0 / np.sqrt(2.0))))
    hn = jnp.concatenate([h, jnp.ones((_Q, 1), jnp.float32)], axis=-1)
    g = lax.dot_general(hn, wg2_ref[...], (((1,), (1,)), ((), ())),
                        precision=_HIGH, preferred_element_type=jnp.float32)
    g = jax.nn.sigmoid(g)
    o_ref[...] = x + g * retrieved


def _sc_gather(table, idx, d):
    bn = idx.shape[0]
    bpw = bn // _NW
    mesh = plsc.VectorSubcoreMesh(core_axis_name="c", subcore_axis_name="s")

    @functools.partial(
        pl.kernel, mesh=mesh,
        out_type=jax.ShapeDtypeStruct((bn, d), table.dtype),
        scratch_types=[
            pltpu.VMEM((bpw,), jnp.int32),
            pltpu.VMEM((bpw, d), table.dtype),
            pltpu.SemaphoreType.DMA,
        ])
    def k(table_hbm, idx_hbm, out_hbm, idx_v, rows_v, sem):
        wid = lax.axis_index("s") * 2 + lax.axis_index("c")
        base = wid * bpw
        pltpu.sync_copy(idx_hbm.at[pl.ds(base, bpw)], idx_v)
        pltpu.async_copy(table_hbm.at[idx_v], rows_v, sem).wait()
        pltpu.sync_copy(rows_v, out_hbm.at[pl.ds(base, bpw)])

    return k(table, idx)


def kernel(x, keys_p, values_p, Wq, Wo, Wg1, bg1, Wg2, bg2):
    x2 = x.reshape(_Q, _D)

    qn = pl.pallas_call(
        _k0_body,
        out_shape=jax.ShapeDtypeStruct((_Q, _D), jnp.bfloat16),
    )(x2, Wq)

    keys_pad = jnp.pad(keys_p, ((0, _MP - _M), (0, 0)))
    sims, gmax = pl.pallas_call(
        _k1_body,
        grid=(_NB,),
        in_specs=[pl.BlockSpec((_Q, _D), lambda i: (0, 0)),
                  pl.BlockSpec((_KB, _D), lambda i: (i, 0))],
        out_specs=[pl.BlockSpec((_Q, _GPB, _G), lambda i: (0, i, 0)),
                   pl.BlockSpec((1, _Q, _GPB), lambda i: (i, 0, 0))],
        out_shape=[jax.ShapeDtypeStruct((_Q, _NG, _G), jnp.float32),
                   jax.ShapeDtypeStruct((_NB, _Q, _GPB), jnp.float32)],
    )(qn, keys_pad)

    gmax_t = gmax.transpose(1, 0, 2).reshape(_Q, _NG)
    gmax_t = jnp.pad(gmax_t, ((0, 0), (0, _NGP - _NG)),
                     constant_values=-2.0)

    pid, flat = pl.pallas_call(
        _k2_body,
        out_shape=[jax.ShapeDtypeStruct((_Q, _K), jnp.int32),
                   jax.ShapeDtypeStruct((_Q, _K), jnp.int32)],
    )(gmax_t)

    gs = _sc_gather(sims.reshape(_Q * _NG, _G), flat.reshape(-1), _G)
    gs = gs.reshape(_Q, _K, _G)

    w8, kidx = pl.pallas_call(
        _k4_body,
        grid=(_Q // _QB4,),
        in_specs=[pl.BlockSpec((_QB4, _K, _G), lambda i: (i, 0, 0)),
                  pl.BlockSpec((_QB4, _K), lambda i: (i, 0))],
        out_specs=[pl.BlockSpec((_QB4, _K), lambda i: (i, 0)),
                   pl.BlockSpec((_QB4, _K), lambda i: (i, 0))],
        out_shape=[jax.ShapeDtypeStruct((_Q, _K), jnp.float32),
                   jax.ShapeDtypeStruct((_Q, _K), jnp.int32)],
    )(gs, pid)

    vals = _sc_gather(values_p, kidx.reshape(-1), _D)
    vals = vals.reshape(_Q, _K, _D)

    wg2a = jnp.concatenate([Wg2, bg2.reshape(1, 1)], axis=1)
    out = pl.pallas_call(
        _k6_body,
        out_shape=jax.ShapeDtypeStruct((_Q, _D), jnp.float32),
    )(x2, vals, w8, Wo, Wg1, bg1.reshape(1, _DH), wg2a)

    return out.reshape(_B, _S, _D)


# speedup vs baseline: 1.9891x; 1.9891x over previous
import functools

import jax
import jax.numpy as jnp
import numpy as np
from jax import lax
from jax.experimental import pallas as pl
from jax.experimental.pallas import tpu as pltpu
from jax.experimental.pallas import tpu_sc as plsc

_B, _S, _D = 4, 512, 128
_Q = _B * _S
_M = 100000
_K = 8
_DH = _D // 2
_KB = 2048
_NB = 49
_MP = _NB * _KB
_G = 128
_GPB = _KB // _G
_NG = _MP // _G
_NGP = 896
_QB4 = 256
_NW = 32

_HIGH = lax.Precision.HIGHEST


def _k0_body(x_ref, wq_ref, qn_ref):
    q = lax.dot_general(x_ref[...], wq_ref[...],
                        (((1,), (1,)), ((), ())),
                        precision=_HIGH, preferred_element_type=jnp.float32)
    ss = jnp.sum(q * q, axis=-1, keepdims=True)
    qn_ref[...] = (q / jnp.maximum(jnp.sqrt(ss), 1e-12)).astype(jnp.bfloat16)


def _k1_body(qn_ref, kb_ref, sims_ref, gmax_ref):
    i = pl.program_id(0)
    kb = kb_ref[...]
    ss = jnp.sum(kb * kb, axis=-1, keepdims=True)
    kn = (kb / jnp.maximum(jnp.sqrt(ss), 1e-12)).astype(jnp.bfloat16)
    s = lax.dot_general(qn_ref[...], kn, (((1,), (1,)), ((), ())),
                        preferred_element_type=jnp.float32)

    gcol = i * _KB + lax.broadcasted_iota(jnp.int32, (_Q, _KB), 1)
    s = jnp.where(gcol < _M, s, jnp.float32(-2.0))
    chunks = []
    for c in range(_GPB):
        sc = s[:, c * _G:(c + 1) * _G]
        sims_ref[:, c * 8:(c + 1) * 8, :] = sc.reshape(_Q // 8, 8, _G)
        chunks.append(jnp.max(sc, axis=-1, keepdims=True))
    gmax_ref[0] = jnp.concatenate(chunks, axis=-1)


def _k2_body(g_ref, pid_ref, flat_ref):
    s = g_ref[...].astype(jnp.float32)
    col = lax.broadcasted_iota(jnp.int32, (_Q, _NGP), 1)
    row = lax.broadcasted_iota(jnp.int32, (_Q, _K), 0)
    ids = []
    for _ in range(_K):
        m = jnp.max(s, axis=-1, keepdims=True)
        cand = jnp.where(s >= m, col, jnp.int32(2 ** 30))
        a = jnp.min(cand, axis=-1, keepdims=True)
        ids.append(a)
        s = jnp.where(col == a, jnp.float32(-3.0), s)
    gid = jnp.concatenate(ids, axis=-1)
    pid_ref[...] = gid
    flat_ref[...] = (row >> 3) * (_NG * 8) + gid * 8 + (row & 7)


def _k4_body(gs_ref, pid_ref, w_ref, idx_ref):
    s3 = gs_ref[...]
    lane = lax.broadcasted_iota(jnp.int32, (_QB4, _K, _G), 2)
    cidx = pid_ref[...][:, :, None] * _G + lane
    vals = []
    idxs = []
    for _ in range(_K):
        m1 = jnp.max(s3, axis=-1)
        mj = jnp.max(m1, axis=-1, keepdims=True)
        mb = mj[:, :, None]
        cand = jnp.where(s3 >= mb, cidx, jnp.int32(2 ** 30))
        a1 = jnp.min(cand, axis=-1)
        a = jnp.min(a1, axis=-1, keepdims=True)
        vals.append(mj)
        idxs.append(a)
        s3 = jnp.where(cidx == a[:, :, None], jnp.float32(-3.0), s3)
    ts = jnp.concatenate(vals, axis=-1)
    mx = jnp.max(ts, axis=-1, keepdims=True)
    e = jnp.exp(ts - mx)
    w_ref[...] = e / jnp.sum(e, axis=-1, keepdims=True)
    idx_ref[...] = jnp.concatenate(idxs, axis=-1)


def _k6_body(x_ref, v_ref, w_ref, wo_ref, wg1_ref, bg1_ref, wg2_ref,
             o_ref):
    x = x_ref[...]
    w = w_ref[...]
    r = w[:, 0:1] * v_ref[:, 0, :]
    for j in range(1, _K):
        r = r + w[:, j:j + 1] * v_ref[:, j, :]
    retrieved = lax.dot_general(r, wo_ref[...], (((1,), (1,)), ((), ())),
                                precision=_HIGH,
                                preferred_element_type=jnp.float32)
    gate_in = jnp.concatenate([x, retrieved], axis=-1)
    h = lax.dot_general(gate_in, wg1_ref[...], (((1,), (1,)), ((), ())),
                        precision=_HIGH, preferred_element_type=jnp.float32)
    h = h + bg1_ref[...]
    h = 0.5 * h * (1.0 + lax.erf(h * np.float32(1.0 / np.sqrt(2.0))))
    hn = jnp.concatenate([h, jnp.ones((_Q, 1), jnp.float32)], axis=-1)
    g = lax.dot_general(hn, wg2_ref[...], (((1,), (1,)), ((), ())),
                        precision=_HIGH, preferred_element_type=jnp.float32)
    g = jax.nn.sigmoid(g)
    o_ref[...] = x + g * retrieved


def _sc_gather(table, idx, d):
    bn = idx.shape[0]
    bpw = bn // _NW
    mesh = plsc.VectorSubcoreMesh(core_axis_name="c", subcore_axis_name="s")

    @functools.partial(
        pl.kernel, mesh=mesh,
        out_type=jax.ShapeDtypeStruct((bn, d), table.dtype),
        scratch_types=[
            pltpu.VMEM((bpw,), jnp.int32),
            pltpu.VMEM((bpw, d), table.dtype),
            pltpu.SemaphoreType.DMA,
        ])
    def k(table_hbm, idx_hbm, out_hbm, idx_v, rows_v, sem):
        wid = lax.axis_index("s") * 2 + lax.axis_index("c")
        base = wid * bpw
        pltpu.sync_copy(idx_hbm.at[pl.ds(base, bpw)], idx_v)
        pltpu.async_copy(table_hbm.at[idx_v], rows_v, sem).wait()
        pltpu.sync_copy(rows_v, out_hbm.at[pl.ds(base, bpw)])

    return k(table, idx)


def kernel(x, keys_p, values_p, Wq, Wo, Wg1, bg1, Wg2, bg2):
    x2 = x.reshape(_Q, _D)

    qn = pl.pallas_call(
        _k0_body,
        out_shape=jax.ShapeDtypeStruct((_Q, _D), jnp.bfloat16),
    )(x2, Wq)

    keys_pad = jnp.pad(keys_p, ((0, _MP - _M), (0, 0)))
    sims, gmax = pl.pallas_call(
        _k1_body,
        grid=(_NB,),
        in_specs=[pl.BlockSpec((_Q, _D), lambda i: (0, 0)),
                  pl.BlockSpec((_KB, _D), lambda i: (i, 0))],
        out_specs=[pl.BlockSpec((_Q // 8, _GPB * 8, _G), lambda i: (0, i, 0)),
                   pl.BlockSpec((1, _Q, _GPB), lambda i: (i, 0, 0))],
        out_shape=[jax.ShapeDtypeStruct((_Q // 8, _NG * 8, _G), jnp.float32),
                   jax.ShapeDtypeStruct((_NB, _Q, _GPB), jnp.float32)],
    )(qn, keys_pad)

    gmax_t = gmax.transpose(1, 0, 2).reshape(_Q, _NG)
    gmax_t = jnp.pad(gmax_t, ((0, 0), (0, _NGP - _NG)),
                     constant_values=-2.0)

    pid, flat = pl.pallas_call(
        _k2_body,
        out_shape=[jax.ShapeDtypeStruct((_Q, _K), jnp.int32),
                   jax.ShapeDtypeStruct((_Q, _K), jnp.int32)],
    )(gmax_t)

    gs = _sc_gather(sims.reshape(_Q * _NG, _G), flat.reshape(-1), _G)
    gs = gs.reshape(_Q, _K, _G)

    w8, kidx = pl.pallas_call(
        _k4_body,
        grid=(_Q // _QB4,),
        in_specs=[pl.BlockSpec((_QB4, _K, _G), lambda i: (i, 0, 0)),
                  pl.BlockSpec((_QB4, _K), lambda i: (i, 0))],
        out_specs=[pl.BlockSpec((_QB4, _K), lambda i: (i, 0)),
                   pl.BlockSpec((_QB4, _K), lambda i: (i, 0))],
        out_shape=[jax.ShapeDtypeStruct((_Q, _K), jnp.float32),
                   jax.ShapeDtypeStruct((_Q, _K), jnp.int32)],
    )(gs, pid)

    vals = _sc_gather(values_p, kidx.reshape(-1), _D)
    vals = vals.reshape(_Q, _K, _D)

    wg2a = jnp.concatenate([Wg2, bg2.reshape(1, 1)], axis=1)
    out = pl.pallas_call(
        _k6_body,
        out_shape=jax.ShapeDtypeStruct((_Q, _D), jnp.float32),
    )(x2, vals, w8, Wo, Wg1, bg1.reshape(1, _DH), wg2a)

    return out.reshape(_B, _S, _D)
